# Initial kernel scaffold; baseline (speedup 1.0000x reference)
import jax
import jax.numpy as jnp
from jax.experimental import pallas as pl


def kernel(features, coors, coors_inv, scale_coors_inv, W_in, b_in, W_pp1, b_pp1, g1, be1, W_pp2, b_pp2, g2, be2, W_pp3, b_pp3, W_out1, b_out1, W_out2, b_out2):
    raise NotImplementedError("write your pallas kernel here")


# speedup vs baseline: 1.1546x; 1.1546x over previous
import functools

import jax
import jax.numpy as jnp
from jax.experimental import pallas as pl
from jax.experimental.pallas import tpu as pltpu

N_VOX = 100000
N_PTS = 400000
N_COARSE = 25000
C = 128
KEYSPACE = 1 << 21

BR = 2000
GRID = N_VOX // BR


def _leaky(x):
    return jnp.where(x >= 0, x, 0.1 * x)


def _p1_body(down_ref, feat_ref, Wpp1_ref, bpp1_ref, Win_ref, bin_ref, W1a_ref,
             h1_ref, A_ref, stats_ref):
    i = pl.program_id(0)
    h1 = _leaky(jnp.dot(down_ref[...], Wpp1_ref[...],
                        preferred_element_type=jnp.float32) + bpp1_ref[...])
    h1_ref[...] = h1
    idn = _leaky(jnp.dot(feat_ref[...], Win_ref[...],
                         preferred_element_type=jnp.float32) + bin_ref[...])
    A_ref[...] = jnp.dot(idn, W1a_ref[...], preferred_element_type=jnp.float32)
    m = jnp.sum(h1, axis=0, keepdims=True)
    q = jnp.sum(h1 * h1, axis=0, keepdims=True)
    blk = jnp.concatenate([m, q], axis=0)

    @pl.when(i == 0)
    def _():
        stats_ref[...] = jnp.zeros_like(stats_ref)

    stats_ref[...] += blk


def _p2_body(h1_ref, W2_ref, b2_ref, h2_ref, stats_ref):
    i = pl.program_id(0)
    h2 = _leaky(jnp.dot(h1_ref[...], W2_ref[...],
                        preferred_element_type=jnp.float32) + b2_ref[...])
    h2_ref[...] = h2
    m = jnp.sum(h2, axis=0, keepdims=True)
    q = jnp.sum(h2 * h2, axis=0, keepdims=True)
    blk = jnp.concatenate([m, q], axis=0)

    @pl.when(i == 0)
    def _():
        stats_ref[...] = jnp.zeros_like(stats_ref)

    stats_ref[...] += blk


def _p3_body(h2_ref, W3_ref, b3_ref, W1b_ref, B_ref):
    h3 = _leaky(jnp.dot(h2_ref[...], W3_ref[...],
                        preferred_element_type=jnp.float32) + b3_ref[...])
    B_ref[...] = jnp.dot(h3, W1b_ref[...], preferred_element_type=jnp.float32)


def _p4_body(A_ref, D_ref, bo1_ref, Wo2_ref, bo2_ref, y_ref):
    pre = _leaky(A_ref[...] + D_ref[...] + bo1_ref[...])
    y_ref[...] = jnp.dot(pre, Wo2_ref[...],
                         preferred_element_type=jnp.float32) + bo2_ref[...]


def _row_block(j=None):
    if j is None:
        return pl.BlockSpec((BR, C), lambda i: (i, 0))
    return pl.BlockSpec((BR, j), lambda i: (i, 0))


def _full(shape):
    return pl.BlockSpec(shape, lambda i: tuple(0 for _ in shape))


def kernel(features, coors, coors_inv, scale_coors_inv, W_in, b_in, W_pp1, b_pp1,
           g1, be1, W_pp2, b_pp2, g2, be2, W_pp3, b_pp3, W_out1, b_out1,
           W_out2, b_out2):
    f32 = jnp.float32
    H = C // 2

    key = (coors[:, 0] << 15) | ((coors[:, 1] >> 1) << 10) \
        | ((coors[:, 2] >> 1) << 5) | (coors[:, 3] >> 1)
    present = jnp.zeros((KEYSPACE,), jnp.int32).at[key].set(
        1, mode="drop", unique_indices=False)
    rank_incl = jnp.cumsum(present)
    n_valid_i = rank_incl[-1]
    inv = rank_incl[key] - 1
    n_valid = n_valid_i.astype(f32)

    cnt = jnp.zeros((N_VOX,), f32).at[inv].add(1.0, mode="drop")
    dsum = jnp.zeros((N_VOX, C), f32).at[inv].add(features, mode="drop")
    down = dsum * (1.0 / jnp.maximum(cnt, 1.0))[:, None]

    W1a = W_out1[:C]
    W1b = W_out1[C:]
    b_pp1r = b_pp1.reshape(1, H)
    b_inr = b_in.reshape(1, C)

    h1, A, st1 = pl.pallas_call(
        _p1_body,
        grid=(GRID,),
        in_specs=[_row_block(), _row_block(), _full((C, H)), _full((1, H)),
                  _full((C, C)), _full((1, C)), _full((C, C))],
        out_specs=[_row_block(H), _row_block(), _full((2, H))],
        out_shape=[jax.ShapeDtypeStruct((N_VOX, H), f32),
                   jax.ShapeDtypeStruct((N_VOX, C), f32),
                   jax.ShapeDtypeStruct((2, H), f32)],
        compiler_params=pltpu.CompilerParams(
            dimension_semantics=("arbitrary",)),
    )(down, features, W_pp1, b_pp1r, W_in, b_inr, W1a)

    n_empty = jnp.float32(N_VOX) - n_valid
    e1 = _leaky(b_pp1)
    m1 = (st1[0] - n_empty * e1) / n_valid
    q1 = (st1[1] - n_empty * e1 * e1) / n_valid
    a1 = g1 / jnp.sqrt(jnp.maximum(q1 - m1 * m1, 0.0) + 1e-5)
    c1 = be1 - m1 * a1
    W2f = a1[:, None] * W_pp2
    b2f = (c1 @ W_pp2 + b_pp2).reshape(1, H)

    h2, st2 = pl.pallas_call(
        _p2_body,
        grid=(GRID,),
        in_specs=[_row_block(H), _full((H, H)), _full((1, H))],
        out_specs=[_row_block(H), _full((2, H))],
        out_shape=[jax.ShapeDtypeStruct((N_VOX, H), f32),
                   jax.ShapeDtypeStruct((2, H), f32)],
        compiler_params=pltpu.CompilerParams(
            dimension_semantics=("arbitrary",)),
    )(h1, W2f, b2f)

    e2 = _leaky(b2f[0])
    m2 = (st2[0] - n_empty * e2) / n_valid
    q2 = (st2[1] - n_empty * e2 * e2) / n_valid
    a2 = g2 / jnp.sqrt(jnp.maximum(q2 - m2 * m2, 0.0) + 1e-5)
    c2 = be2 - m2 * a2
    W3f = a2[:, None] * W_pp3
    b3f = (c2 @ W_pp3 + b_pp3).reshape(1, C)

    B = pl.pallas_call(
        _p3_body,
        grid=(GRID,),
        in_specs=[_row_block(H), _full((H, C)), _full((1, C)), _full((C, C))],
        out_specs=_row_block(),
        out_shape=jax.ShapeDtypeStruct((N_VOX, C), f32),
        compiler_params=pltpu.CompilerParams(
            dimension_semantics=("arbitrary",)),
    )(h2, W3f, b3f, W1b)

    D = B[inv]
    y = pl.pallas_call(
        _p4_body,
        grid=(GRID,),
        in_specs=[_row_block(), _row_block(), _full((1, C)), _full((C, C)),
                  _full((1, C))],
        out_specs=_row_block(),
        out_shape=jax.ShapeDtypeStruct((N_VOX, C), f32),
        compiler_params=pltpu.CompilerParams(
            dimension_semantics=("arbitrary",)),
    )(A, D, b_out1.reshape(1, C), W_out2, b_out2.reshape(1, C))

    z = y[coors_inv]
    cnt2 = jnp.zeros((N_COARSE,), f32).at[scale_coors_inv].add(1.0, mode="drop")
    num = jnp.zeros((N_COARSE, C), f32).at[scale_coors_inv].add(z, mode="drop")
    v_feat = num * (1.0 / jnp.maximum(cnt2, 1.0))[:, None]
    return v_feat


# speedup vs baseline: 2.0509x; 1.7763x over previous
import functools

import jax
import jax.numpy as jnp
from jax import lax
from jax.experimental import pallas as pl
from jax.experimental.pallas import tpu as pltpu
from jax.experimental.pallas import tpu_sc as plsc

N_VOX = 100000
N_PTS = 400000
N_COARSE = 25000
C = 128
H = C // 2
KEYSPACE = 1 << 21
KR = KEYSPACE // 32

BR = 2000
GRID = N_VOX // BR

NS = 16

VCH = 49
VPT = 6250
KCH = 25
KPT = 3125
PCH = 196
PPT = 25000
PACC = PCH * 128
DUMP3 = N_COARSE

f32 = jnp.float32
i32 = jnp.int32


def _leaky(x):
    return jnp.where(x >= 0, x, 0.1 * x)


def _p1_body(down_ref, feat_ref, Wpp1_ref, bpp1_ref, Win_ref, bin_ref,
             W1a_ref, h1_ref, A_ref, stats_ref):
    i = pl.program_id(0)
    h1 = _leaky(jnp.dot(down_ref[...], Wpp1_ref[...],
                        preferred_element_type=f32) + bpp1_ref[...])
    h1_ref[...] = h1
    idn = _leaky(jnp.dot(feat_ref[...], Win_ref[...],
                         preferred_element_type=f32) + bin_ref[...])
    A_ref[...] = jnp.dot(idn, W1a_ref[...], preferred_element_type=f32)
    blk = jnp.concatenate([jnp.sum(h1, axis=0, keepdims=True),
                           jnp.sum(h1 * h1, axis=0, keepdims=True)], axis=0)

    @pl.when(i == 0)
    def _():
        stats_ref[...] = jnp.zeros_like(stats_ref)

    stats_ref[...] += blk


def _p2_body(h1_ref, W2_ref, b2_ref, h2_ref, stats_ref):
    i = pl.program_id(0)
    h2 = _leaky(jnp.dot(h1_ref[...], W2_ref[...],
                        preferred_element_type=f32) + b2_ref[...])
    h2_ref[...] = h2
    blk = jnp.concatenate([jnp.sum(h2, axis=0, keepdims=True),
                           jnp.sum(h2 * h2, axis=0, keepdims=True)], axis=0)

    @pl.when(i == 0)
    def _():
        stats_ref[...] = jnp.zeros_like(stats_ref)

    stats_ref[...] += blk


def _p3_body(h2_ref, W3_ref, b3_ref, W1b_ref, B_ref):
    h3 = _leaky(jnp.dot(h2_ref[...], W3_ref[...],
                        preferred_element_type=f32) + b3_ref[...])
    B_ref[...] = jnp.dot(h3, W1b_ref[...], preferred_element_type=f32)


def _p4_body(A_ref, D_ref, bo1_ref, Wo2_ref, bo2_ref, y_ref):
    pre = _leaky(A_ref[...] + D_ref[...] + bo1_ref[...])
    y_ref[...] = jnp.dot(pre, Wo2_ref[...],
                         preferred_element_type=f32) + bo2_ref[...]


def _rows(j=C):
    return pl.BlockSpec((BR, j), lambda i: (i, 0))


def _full(shape):
    return pl.BlockSpec(shape, lambda i: tuple(0 for _ in shape))


_SEQ = pltpu.CompilerParams(dimension_semantics=("arbitrary",))


_MESH = plsc.VectorSubcoreMesh(core_axis_name="c", subcore_axis_name="s")
_SC_PARAMS = pltpu.CompilerParams(use_tc_tiling_on_sc=False,
                                  needs_layout_passes=False)


def _fill2d(ref, nrows, ncols, val):
    nv = ncols // 16

    def body(i, carry):
        r = i // nv
        k = i % nv
        ref[r, pl.ds(k * 16, 16)] = jnp.full((16,), val, f32)
        return carry

    lax.fori_loop(0, nrows * nv, body, 0)


@functools.partial(
    pl.kernel,
    out_type=[jax.ShapeDtypeStruct((32, NS, VCH, 128), i32),
              jax.ShapeDtypeStruct((32, 16), i32)],
    mesh=_MESH,
    compiler_params=_SC_PARAMS,
    scratch_types=[
        pltpu.VMEM((KR,), i32),
        pltpu.VMEM((VCH, 128), i32),
        pltpu.VMEM((VCH, 128), i32),
        pltpu.VMEM((16,), i32),
        pltpu.SemaphoreType.DMA,
    ],
)
def _k0(key3, part_out, tot_out, table_v, keyb_v, outb_v, tot_v, sem):
    c = lax.axis_index("c")
    s = lax.axis_index("s")
    w = c * NS + s
    lo = w * KR

    def zb(idx, carry):
        table_v[pl.ds(idx * 16, 16)] = jnp.zeros((16,), i32)
        return carry

    lax.fori_loop(0, KR // 16, zb, 0)

    ones16 = jnp.ones((16,), i32)

    def tpass(t, carry):
        pltpu.sync_copy(key3.at[t], keyb_v)

        def jloop(idx, carry2):
            j = idx // 8
            k = idx % 8
            vec = keyb_v[j, pl.ds(k * 16, 16)]
            rel = vec - lo
            m = (rel >= 0) & (rel < KR)
            plsc.store_scatter(table_v, [rel], ones16, mask=m)
            return carry2

        lax.fori_loop(0, VCH * 8, jloop, 0)
        return carry

    lax.fori_loop(0, NS, tpass, 0)

    def scan(idx, carry):
        v = table_v[pl.ds(idx * 16, 16)]
        inc = plsc.cumsum(v)
        table_v[pl.ds(idx * 16, 16)] = inc - v + carry
        return carry + jnp.sum(v)

    tot = lax.fori_loop(0, KR // 16, scan, i32(0))

    def qpass(t, carry):
        pltpu.sync_copy(key3.at[t], keyb_v)

        def jloop(idx, carry2):
            j = idx // 8
            k = idx % 8
            vec = keyb_v[j, pl.ds(k * 16, 16)]
            rel = vec - lo
            m = (rel >= 0) & (rel < KR)
            g = plsc.load_gather(table_v, [rel], mask=m)
            outb_v[j, pl.ds(k * 16, 16)] = jnp.where(m, g + 1, 0)
            return carry2

        lax.fori_loop(0, VCH * 8, jloop, 0)
        pltpu.sync_copy(outb_v, part_out.at[w, t])
        return carry

    lax.fori_loop(0, NS, qpass, 0)
    tot_v[pl.ds(0, 16)] = jnp.zeros((16,), i32) + tot
    pltpu.sync_copy(tot_v, tot_out.at[w])


@functools.partial(
    pl.kernel,
    out_type=jax.ShapeDtypeStruct((N_VOX, C), f32),
    mesh=_MESH,
    compiler_params=_SC_PARAMS,
    scratch_types=[
        pltpu.VMEM((KCH, 128), i32),
        pltpu.VMEM((128, 128), f32),
        pltpu.SemaphoreType.DMA,
    ],
)
def _k2(B, invK, D_out, idx_v, rows_v, sem):
    c = lax.axis_index("c")
    s = lax.axis_index("s")
    w = c * NS + s
    base = w * KPT
    pltpu.sync_copy(invK.at[c, s], idx_v)

    def chunk(j, carry):
        pltpu.async_copy(B.at[idx_v.at[j]], rows_v, sem).wait()
        pltpu.sync_copy(rows_v, D_out.at[pl.ds(base + j * 128, 128)])
        return carry

    lax.fori_loop(0, KCH - 1, chunk, 0)
    pltpu.async_copy(B.at[idx_v.at[KCH - 1]], rows_v, sem).wait()
    pltpu.sync_copy(rows_v.at[pl.ds(0, 53)],
                    D_out.at[pl.ds(base + (KCH - 1) * 128, 53)])


@functools.partial(
    pl.kernel,
    out_type=jax.ShapeDtypeStruct((4, PACC, 32), f32),
    mesh=_MESH,
    compiler_params=_SC_PARAMS,
    scratch_types=[
        pltpu.VMEM((PCH, 128), i32),
        pltpu.VMEM((PCH, 128), i32),
        pltpu.VMEM((128, 32), f32),
        pltpu.VMEM((56, 32), f32),
        pltpu.VMEM_SHARED((PACC, 32), f32),
        pltpu.SemaphoreType.DMA,
    ],
)
def _k3(y4, gidx4, sidx, num_out,
        gidx_v, sidx_v, rows_v, z32_v, acc_sh, sem):
    c = lax.axis_index("c")
    s = lax.axis_index("s")
    spt = PACC // NS

    _fill2d(z32_v, 56, 32, 0.0)
    pltpu.sync_copy(sidx.at[s], sidx_v)

    for q in range(2):
        pltpu.sync_copy(gidx4.at[c, q, s], gidx_v)

        def zchunk(j, carry):
            pltpu.sync_copy(z32_v, acc_sh.at[pl.ds(s * spt + j * 56, 56)])
            return carry

        lax.fori_loop(0, spt // 56, zchunk, 0)
        plsc.subcore_barrier()

        def chunk(j, carry):
            pltpu.async_copy(y4.at[gidx_v.at[j]], rows_v, sem).wait()
            pltpu.sync_copy(rows_v, acc_sh.at[sidx_v.at[j]], add=True)
            return carry

        lax.fori_loop(0, PCH, chunk, 0)
        plsc.subcore_barrier()

        pltpu.sync_copy(acc_sh.at[pl.ds(s * spt, spt)],
                        num_out.at[2 * c + q, pl.ds(s * spt, spt)])


def kernel(features, coors, coors_inv, scale_coors_inv, W_in, b_in, W_pp1,
           b_pp1, g1, be1, W_pp2, b_pp2, g2, be2, W_pp3, b_pp3, W_out1,
           b_out1, W_out2, b_out2):
    key = (coors[:, 0] << 15) | ((coors[:, 1] >> 1) << 10) \
        | ((coors[:, 2] >> 1) << 5) | (coors[:, 3] >> 1)
    key3 = jnp.pad(key.reshape(NS, VPT), ((0, 0), (0, VCH * 128 - VPT)),
                   mode="edge").reshape(NS, VCH, 128)
    part, totals = _k0(key3)
    tot = totals[:, 0]
    offs = jnp.concatenate([jnp.zeros((1,), i32), jnp.cumsum(tot)])[:32]
    psum = jnp.sum(part, axis=0).reshape(NS, VCH * 128)[:, :VPT].reshape(-1)
    inv = psum - 1 + offs[key >> 16]
    n_valid = jnp.sum(tot).astype(f32)

    invK = jnp.pad(inv.reshape(32, KPT), ((0, 0), (0, KCH * 128 - KPT)),
                   constant_values=0).reshape(2, NS, KCH, 128)
    base4 = 4 * jnp.pad(coors_inv.reshape(NS, PPT),
                        ((0, 0), (0, PACC - PPT)),
                        constant_values=0).reshape(NS, PCH, 128)
    gidx4 = jnp.stack([jnp.stack([base4, base4 + 1]),
                       jnp.stack([base4 + 2, base4 + 3])])
    sidx = jnp.pad(scale_coors_inv.reshape(NS, PPT),
                   ((0, 0), (0, PACC - PPT)),
                   constant_values=DUMP3).reshape(NS, PCH, 128)
    cnt2 = jnp.zeros((N_COARSE,), f32).at[scale_coors_inv].add(
        1.0, mode="drop")

    cnt = jnp.zeros((N_VOX,), f32).at[inv].add(1.0, mode="drop")
    dsum = jnp.zeros((N_VOX, C), f32).at[inv].add(features, mode="drop")
    down = dsum * (1.0 / jnp.maximum(cnt, 1.0))[:, None]

    W1a = W_out1[:C]
    W1b = W_out1[C:]

    h1, A, st1 = pl.pallas_call(
        _p1_body,
        grid=(GRID,),
        in_specs=[_rows(), _rows(), _full((C, H)), _full((1, H)),
                  _full((C, C)), _full((1, C)), _full((C, C))],
        out_specs=[_rows(H), _rows(), _full((2, H))],
        out_shape=[jax.ShapeDtypeStruct((N_VOX, H), f32),
                   jax.ShapeDtypeStruct((N_VOX, C), f32),
                   jax.ShapeDtypeStruct((2, H), f32)],
        compiler_params=_SEQ,
    )(down, features, W_pp1, b_pp1.reshape(1, H), W_in,
      b_in.reshape(1, C), W1a)

    n_empty = jnp.float32(N_VOX) - n_valid
    e1 = _leaky(b_pp1)
    m1 = (st1[0] - n_empty * e1) / n_valid
    q1 = (st1[1] - n_empty * e1 * e1) / n_valid
    a1 = g1 / jnp.sqrt(jnp.maximum(q1 - m1 * m1, 0.0) + 1e-5)
    c1 = be1 - m1 * a1
    W2f = a1[:, None] * W_pp2
    b2f = (c1 @ W_pp2 + b_pp2).reshape(1, H)

    h2, st2 = pl.pallas_call(
        _p2_body,
        grid=(GRID,),
        in_specs=[_rows(H), _full((H, H)), _full((1, H))],
        out_specs=[_rows(H), _full((2, H))],
        out_shape=[jax.ShapeDtypeStruct((N_VOX, H), f32),
                   jax.ShapeDtypeStruct((2, H), f32)],
        compiler_params=_SEQ,
    )(h1, W2f, b2f)

    e2 = _leaky(b2f[0])
    m2 = (st2[0] - n_empty * e2) / n_valid
    q2 = (st2[1] - n_empty * e2 * e2) / n_valid
    a2 = g2 / jnp.sqrt(jnp.maximum(q2 - m2 * m2, 0.0) + 1e-5)
    c2 = be2 - m2 * a2
    W3f = a2[:, None] * W_pp3
    b3f = (c2 @ W_pp3 + b_pp3).reshape(1, C)

    B = pl.pallas_call(
        _p3_body,
        grid=(GRID,),
        in_specs=[_rows(H), _full((H, C)), _full((1, C)), _full((C, C))],
        out_specs=_rows(),
        out_shape=jax.ShapeDtypeStruct((N_VOX, C), f32),
        compiler_params=_SEQ,
    )(h2, W3f, b3f, W1b)

    D = _k2(B, invK)

    y = pl.pallas_call(
        _p4_body,
        grid=(GRID,),
        in_specs=[_rows(), _rows(), _full((1, C)), _full((C, C)),
                  _full((1, C))],
        out_specs=_rows(),
        out_shape=jax.ShapeDtypeStruct((N_VOX, C), f32),
        compiler_params=_SEQ,
    )(A, D, b_out1.reshape(1, C), W_out2, b_out2.reshape(1, C))

    num = _k3(y.reshape(4 * N_VOX, 32), gidx4, sidx)
    scale = 1.0 / jnp.maximum(cnt2, 1.0)
    v_feat = jnp.concatenate(
        [num[0, :N_COARSE], num[1, :N_COARSE],
         num[2, :N_COARSE], num[3, :N_COARSE]], axis=1) * scale[:, None]
    return v_feat


# speedup vs baseline: 2.5696x; 1.2529x over previous
import functools

import jax
import jax.numpy as jnp
from jax import lax
from jax.experimental import pallas as pl
from jax.experimental.pallas import tpu as pltpu
from jax.experimental.pallas import tpu_sc as plsc

N_VOX = 100000
N_PTS = 400000
N_COARSE = 25000
C = 128
H = C // 2
KEYSPACE = 1 << 21
KR = KEYSPACE // 32

BR = 2000
GRID = N_VOX // BR

NS = 16

VCH = 49
VPT = 6250
KCH = 25
KPT = 3125
PCH = 196
PPT = 25000
PACC = PCH * 128
DUMP3 = N_COARSE

f32 = jnp.float32
i32 = jnp.int32


def _leaky(x):
    return jnp.where(x >= 0, x, 0.1 * x)


def _p1_body(dsx_ref, feat_ref, Wpp1_ref, bpp1_ref, Win_ref, bin_ref,
             W1a_ref, h1_ref, A_ref, stats_ref):
    i = pl.program_id(0)
    x = dsx_ref[...]
    rc = jnp.maximum(x[:, C:C + 1], 1.0)
    h1 = _leaky(jnp.dot(x[:, :C], Wpp1_ref[...],
                        preferred_element_type=f32) / rc + bpp1_ref[...])
    h1_ref[...] = h1
    idn = _leaky(jnp.dot(feat_ref[...], Win_ref[...],
                         preferred_element_type=f32) + bin_ref[...])
    A_ref[...] = jnp.dot(idn, W1a_ref[...], preferred_element_type=f32)
    blk = jnp.concatenate([jnp.sum(h1, axis=0, keepdims=True),
                           jnp.sum(h1 * h1, axis=0, keepdims=True)], axis=0)

    @pl.when(i == 0)
    def _():
        stats_ref[...] = jnp.zeros_like(stats_ref)

    stats_ref[...] += blk


def _p2_body(h1_ref, W2_ref, b2_ref, h2_ref, stats_ref):
    i = pl.program_id(0)
    h2 = _leaky(jnp.dot(h1_ref[...], W2_ref[...],
                        preferred_element_type=f32) + b2_ref[...])
    h2_ref[...] = h2
    blk = jnp.concatenate([jnp.sum(h2, axis=0, keepdims=True),
                           jnp.sum(h2 * h2, axis=0, keepdims=True)], axis=0)

    @pl.when(i == 0)
    def _():
        stats_ref[...] = jnp.zeros_like(stats_ref)

    stats_ref[...] += blk


def _p3_body(h2_ref, W3_ref, b3_ref, W1b_ref, B_ref):
    h3 = _leaky(jnp.dot(h2_ref[...], W3_ref[...],
                        preferred_element_type=f32) + b3_ref[...])
    B_ref[...] = jnp.dot(h3, W1b_ref[...], preferred_element_type=f32)


def _p4_body(A_ref, D_ref, bo1_ref, Wo2_ref, bo2_ref, y_ref):
    pre = _leaky(A_ref[...] + D_ref[...] + bo1_ref[...])
    y_ref[...] = jnp.dot(pre, Wo2_ref[...],
                         preferred_element_type=f32) + bo2_ref[...]


def _rows(j=C):
    return pl.BlockSpec((BR, j), lambda i: (i, 0))


def _full(shape):
    return pl.BlockSpec(shape, lambda i: tuple(0 for _ in shape))


_SEQ = pltpu.CompilerParams(dimension_semantics=("arbitrary",))


_MESH = plsc.VectorSubcoreMesh(core_axis_name="c", subcore_axis_name="s")
_SC_PARAMS = pltpu.CompilerParams(use_tc_tiling_on_sc=False,
                                  needs_layout_passes=False)


def _fill2d(ref, nrows, ncols, val):
    nv = ncols // 16

    def body(i, carry):
        r = i // nv
        k = i % nv
        ref[r, pl.ds(k * 16, 16)] = jnp.full((16,), val, f32)
        return carry

    lax.fori_loop(0, nrows * nv, body, 0)


@functools.partial(
    pl.kernel,
    out_type=[jax.ShapeDtypeStruct((32, NS, VCH, 128), i32),
              jax.ShapeDtypeStruct((32, 16), i32)],
    mesh=_MESH,
    compiler_params=_SC_PARAMS,
    scratch_types=[
        pltpu.VMEM((KR,), i32),
        pltpu.VMEM((VCH, 128), i32),
        pltpu.VMEM((VCH, 128), i32),
        pltpu.VMEM((16,), i32),
        pltpu.SemaphoreType.DMA,
    ],
)
def _k0(key3, part_out, tot_out, table_v, keyb_v, outb_v, tot_v, sem):
    c = lax.axis_index("c")
    s = lax.axis_index("s")
    w = c * NS + s
    lo = w * KR

    def zb(idx, carry):
        table_v[pl.ds(idx * 16, 16)] = jnp.zeros((16,), i32)
        return carry

    lax.fori_loop(0, KR // 16, zb, 0)

    ones16 = jnp.ones((16,), i32)

    def tpass(t, carry):
        pltpu.sync_copy(key3.at[t], keyb_v)

        def jloop(idx, carry2):
            j = idx // 8
            k = idx % 8
            vec = keyb_v[j, pl.ds(k * 16, 16)]
            rel = vec - lo
            m = (rel >= 0) & (rel < KR)
            plsc.store_scatter(table_v, [rel], ones16, mask=m)
            return carry2

        lax.fori_loop(0, VCH * 8, jloop, 0)
        return carry

    lax.fori_loop(0, NS, tpass, 0)

    def scan(idx, carry):
        v = table_v[pl.ds(idx * 16, 16)]
        inc = plsc.cumsum(v)
        table_v[pl.ds(idx * 16, 16)] = inc - v + carry
        return carry + jnp.sum(v)

    tot = lax.fori_loop(0, KR // 16, scan, i32(0))

    def qpass(t, carry):
        pltpu.sync_copy(key3.at[t], keyb_v)

        def jloop(idx, carry2):
            j = idx // 8
            k = idx % 8
            vec = keyb_v[j, pl.ds(k * 16, 16)]
            rel = vec - lo
            m = (rel >= 0) & (rel < KR)
            g = plsc.load_gather(table_v, [rel], mask=m)
            outb_v[j, pl.ds(k * 16, 16)] = jnp.where(m, g + 1, 0)
            return carry2

        lax.fori_loop(0, VCH * 8, jloop, 0)
        pltpu.sync_copy(outb_v, part_out.at[w, t])
        return carry

    lax.fori_loop(0, NS, qpass, 0)
    tot_v[pl.ds(0, 16)] = jnp.zeros((16,), i32) + tot
    pltpu.sync_copy(tot_v, tot_out.at[w])


@functools.partial(
    pl.kernel,
    out_type=jax.ShapeDtypeStruct((N_VOX, C), f32),
    mesh=_MESH,
    compiler_params=_SC_PARAMS,
    scratch_types=[
        pltpu.VMEM((KCH, 128), i32),
        pltpu.VMEM((128, 128), f32),
        pltpu.VMEM((128, 128), f32),
        pltpu.SemaphoreType.DMA,
        pltpu.SemaphoreType.DMA,
    ],
)
def _k2(B, invK, D_out, idx_v, rows_v, rows2_v, sem, sem2):
    c = lax.axis_index("c")
    s = lax.axis_index("s")
    w = c * NS + s
    base = w * KPT
    pltpu.sync_copy(invK.at[c, s], idx_v)

    nfull = KCH - 1
    pltpu.async_copy(B.at[idx_v.at[0]], rows_v, sem)

    def spair(jj, carry):
        j = 2 * jj
        pltpu.async_copy(B.at[idx_v.at[j + 1]], rows2_v, sem2)
        pltpu.make_async_copy(B.at[idx_v.at[j]], rows_v, sem).wait()
        pltpu.sync_copy(rows_v, D_out.at[pl.ds(base + j * 128, 128)])

        @pl.when(j + 2 < nfull)
        def _():
            pltpu.async_copy(B.at[idx_v.at[j + 2]], rows_v, sem)

        pltpu.make_async_copy(B.at[idx_v.at[j + 1]], rows2_v, sem2).wait()
        pltpu.sync_copy(rows2_v, D_out.at[pl.ds(base + (j + 1) * 128, 128)])
        return carry

    lax.fori_loop(0, nfull // 2, spair, 0)
    pltpu.async_copy(B.at[idx_v.at[KCH - 1]], rows_v, sem).wait()
    pltpu.sync_copy(rows_v.at[pl.ds(0, 53)],
                    D_out.at[pl.ds(base + (KCH - 1) * 128, 53)])


@functools.partial(
    pl.kernel,
    out_type=jax.ShapeDtypeStruct((4, PACC, 32), f32),
    mesh=_MESH,
    compiler_params=_SC_PARAMS,
    scratch_types=[
        pltpu.VMEM((PCH, 128), i32),
        pltpu.VMEM((PCH, 128), i32),
        pltpu.VMEM((128, 32), f32),
        pltpu.VMEM((128, 32), f32),
        pltpu.VMEM((56, 32), f32),
        pltpu.VMEM_SHARED((PACC, 32), f32),
        pltpu.SemaphoreType.DMA,
        pltpu.SemaphoreType.DMA,
    ],
)
def _k3(y4, gidx4, sidx, num_out,
        gidx_v, sidx_v, rows_v, rows2_v, z32_v, acc_sh, sem, sem2):
    c = lax.axis_index("c")
    s = lax.axis_index("s")
    spt = PACC // NS

    _fill2d(z32_v, 56, 32, 0.0)
    pltpu.sync_copy(sidx.at[s], sidx_v)

    for q in range(2):
        pltpu.sync_copy(gidx4.at[c, q, s], gidx_v)

        def zchunk(j, carry):
            pltpu.sync_copy(z32_v, acc_sh.at[pl.ds(s * spt + j * 56, 56)])
            return carry

        lax.fori_loop(0, spt // 56, zchunk, 0)
        plsc.subcore_barrier()

        pltpu.async_copy(y4.at[gidx_v.at[0]], rows_v, sem)

        def spair(jj, carry):
            j = 2 * jj
            pltpu.async_copy(y4.at[gidx_v.at[j + 1]], rows2_v, sem2)
            pltpu.make_async_copy(y4.at[gidx_v.at[j]], rows_v, sem).wait()
            pltpu.sync_copy(rows_v, acc_sh.at[sidx_v.at[j]], add=True)

            @pl.when(j + 2 < PCH)
            def _():
                pltpu.async_copy(y4.at[gidx_v.at[j + 2]], rows_v, sem)

            pltpu.make_async_copy(y4.at[gidx_v.at[j + 1]], rows2_v,
                                  sem2).wait()
            pltpu.sync_copy(rows2_v, acc_sh.at[sidx_v.at[j + 1]], add=True)
            return carry

        lax.fori_loop(0, PCH // 2, spair, 0)
        plsc.subcore_barrier()

        pltpu.sync_copy(acc_sh.at[pl.ds(s * spt, spt)],
                        num_out.at[2 * c + q, pl.ds(s * spt, spt)])


def kernel(features, coors, coors_inv, scale_coors_inv, W_in, b_in, W_pp1,
           b_pp1, g1, be1, W_pp2, b_pp2, g2, be2, W_pp3, b_pp3, W_out1,
           b_out1, W_out2, b_out2):
    key = (coors[:, 0] << 15) | ((coors[:, 1] >> 1) << 10) \
        | ((coors[:, 2] >> 1) << 5) | (coors[:, 3] >> 1)
    key3 = jnp.pad(key.reshape(NS, VPT), ((0, 0), (0, VCH * 128 - VPT)),
                   mode="edge").reshape(NS, VCH, 128)
    part, totals = _k0(key3)
    tot = totals[:, 0]
    offs = jnp.concatenate([jnp.zeros((1,), i32), jnp.cumsum(tot)])[:32]
    psum = jnp.sum(part, axis=0).reshape(NS, VCH * 128)[:, :VPT].reshape(-1)
    inv = psum - 1 + offs[key >> 16]
    n_valid = jnp.sum(tot).astype(f32)

    invK = jnp.pad(inv.reshape(32, KPT), ((0, 0), (0, KCH * 128 - KPT)),
                   constant_values=0).reshape(2, NS, KCH, 128)
    base4 = 4 * jnp.pad(coors_inv.reshape(NS, PPT),
                        ((0, 0), (0, PACC - PPT)),
                        constant_values=0).reshape(NS, PCH, 128)
    gidx4 = jnp.stack([jnp.stack([base4, base4 + 1]),
                       jnp.stack([base4 + 2, base4 + 3])])
    sidx = jnp.pad(scale_coors_inv.reshape(NS, PPT),
                   ((0, 0), (0, PACC - PPT)),
                   constant_values=DUMP3).reshape(NS, PCH, 128)
    cnt2 = jnp.zeros((N_COARSE,), f32).at[scale_coors_inv].add(
        1.0, mode="drop")

    fea_ext = jnp.concatenate([features, jnp.ones((N_VOX, 8), f32)], axis=1)
    dsx = jnp.zeros((N_VOX, C + 8), f32).at[inv].add(fea_ext, mode="drop")

    W1a = W_out1[:C]
    W1b = W_out1[C:]

    h1, A, st1 = pl.pallas_call(
        _p1_body,
        grid=(GRID,),
        in_specs=[_rows(C + 8), _rows(), _full((C, H)), _full((1, H)),
                  _full((C, C)), _full((1, C)), _full((C, C))],
        out_specs=[_rows(H), _rows(), _full((2, H))],
        out_shape=[jax.ShapeDtypeStruct((N_VOX, H), f32),
                   jax.ShapeDtypeStruct((N_VOX, C), f32),
                   jax.ShapeDtypeStruct((2, H), f32)],
        compiler_params=_SEQ,
    )(dsx, features, W_pp1, b_pp1.reshape(1, H), W_in,
      b_in.reshape(1, C), W1a)

    n_empty = jnp.float32(N_VOX) - n_valid
    e1 = _leaky(b_pp1)
    m1 = (st1[0] - n_empty * e1) / n_valid
    q1 = (st1[1] - n_empty * e1 * e1) / n_valid
    a1 = g1 / jnp.sqrt(jnp.maximum(q1 - m1 * m1, 0.0) + 1e-5)
    c1 = be1 - m1 * a1
    W2f = a1[:, None] * W_pp2
    b2f = (c1 @ W_pp2 + b_pp2).reshape(1, H)

    h2, st2 = pl.pallas_call(
        _p2_body,
        grid=(GRID,),
        in_specs=[_rows(H), _full((H, H)), _full((1, H))],
        out_specs=[_rows(H), _full((2, H))],
        out_shape=[jax.ShapeDtypeStruct((N_VOX, H), f32),
                   jax.ShapeDtypeStruct((2, H), f32)],
        compiler_params=_SEQ,
    )(h1, W2f, b2f)

    e2 = _leaky(b2f[0])
    m2 = (st2[0] - n_empty * e2) / n_valid
    q2 = (st2[1] - n_empty * e2 * e2) / n_valid
    a2 = g2 / jnp.sqrt(jnp.maximum(q2 - m2 * m2, 0.0) + 1e-5)
    c2 = be2 - m2 * a2
    W3f = a2[:, None] * W_pp3
    b3f = (c2 @ W_pp3 + b_pp3).reshape(1, C)

    B = pl.pallas_call(
        _p3_body,
        grid=(GRID,),
        in_specs=[_rows(H), _full((H, C)), _full((1, C)), _full((C, C))],
        out_specs=_rows(),
        out_shape=jax.ShapeDtypeStruct((N_VOX, C), f32),
        compiler_params=_SEQ,
    )(h2, W3f, b3f, W1b)

    D = _k2(B, invK)

    y = pl.pallas_call(
        _p4_body,
        grid=(GRID,),
        in_specs=[_rows(), _rows(), _full((1, C)), _full((C, C)),
                  _full((1, C))],
        out_specs=_rows(),
        out_shape=jax.ShapeDtypeStruct((N_VOX, C), f32),
        compiler_params=_SEQ,
    )(A, D, b_out1.reshape(1, C), W_out2, b_out2.reshape(1, C))

    num = _k3(y.reshape(4 * N_VOX, 32), gidx4, sidx)
    scale = 1.0 / jnp.maximum(cnt2, 1.0)
    v_feat = jnp.concatenate(
        [num[0, :N_COARSE], num[1, :N_COARSE],
         num[2, :N_COARSE], num[3, :N_COARSE]], axis=1) * scale[:, None]
    return v_feat


# speedup vs baseline: 2.7836x; 1.0833x over previous
import functools

import jax
import jax.numpy as jnp
from jax import lax
from jax.experimental import pallas as pl
from jax.experimental.pallas import tpu as pltpu
from jax.experimental.pallas import tpu_sc as plsc

N_VOX = 100000
N_PTS = 400000
N_COARSE = 25000
C = 128
H = C // 2
KEYSPACE = 1 << 21
KR = KEYSPACE // 32

BR = 2000
GRID = N_VOX // BR

NS = 16

VCH = 49
VPT = 6250
KCH = 25
KPT = 3125
PCH = 196
PPT = 25000
PACC = PCH * 128
DUMP3 = N_COARSE

f32 = jnp.float32
i32 = jnp.int32


def _leaky(x):
    return jnp.where(x >= 0, x, 0.1 * x)


def _p0_body(feat_ref, Wpp1_ref, Win_ref, bin_ref, W1a_ref,
             F1_ref, A_ref):
    F1_ref[...] = jnp.dot(feat_ref[...], Wpp1_ref[...],
                          preferred_element_type=f32)
    idn = _leaky(jnp.dot(feat_ref[...], Win_ref[...],
                         preferred_element_type=f32) + bin_ref[...])
    A_ref[...] = jnp.dot(idn, W1a_ref[...], preferred_element_type=f32)


def _p1_body(dsx_ref, bpp1_ref, h1_ref, stats_ref):
    i = pl.program_id(0)
    x = dsx_ref[...]
    rc = jnp.maximum(x[:, H:H + 1], 1.0)
    h1 = _leaky(x[:, :H] / rc + bpp1_ref[...])
    h1_ref[...] = h1
    blk = jnp.concatenate([jnp.sum(h1, axis=0, keepdims=True),
                           jnp.sum(h1 * h1, axis=0, keepdims=True)], axis=0)

    @pl.when(i == 0)
    def _():
        stats_ref[...] = jnp.zeros_like(stats_ref)

    stats_ref[...] += blk


def _p2_body(h1_ref, W2_ref, b2_ref, h2_ref, stats_ref):
    i = pl.program_id(0)
    h2 = _leaky(jnp.dot(h1_ref[...], W2_ref[...],
                        preferred_element_type=f32) + b2_ref[...])
    h2_ref[...] = h2
    blk = jnp.concatenate([jnp.sum(h2, axis=0, keepdims=True),
                           jnp.sum(h2 * h2, axis=0, keepdims=True)], axis=0)

    @pl.when(i == 0)
    def _():
        stats_ref[...] = jnp.zeros_like(stats_ref)

    stats_ref[...] += blk


def _p3_body(h2_ref, W3_ref, b3_ref, W1b_ref, B_ref):
    h3 = _leaky(jnp.dot(h2_ref[...], W3_ref[...],
                        preferred_element_type=f32) + b3_ref[...])
    B_ref[...] = jnp.dot(h3, W1b_ref[...], preferred_element_type=f32)


def _p4_body(A_ref, D_ref, bo1_ref, Wo2_ref, bo2_ref, y_ref):
    pre = _leaky(A_ref[...] + D_ref[...] + bo1_ref[...])
    y_ref[...] = jnp.dot(pre, Wo2_ref[...],
                         preferred_element_type=f32) + bo2_ref[...]


def _rows(j=C):
    return pl.BlockSpec((BR, j), lambda i: (i, 0))


def _full(shape):
    return pl.BlockSpec(shape, lambda i: tuple(0 for _ in shape))


_SEQ = pltpu.CompilerParams(dimension_semantics=("arbitrary",))


_MESH = plsc.VectorSubcoreMesh(core_axis_name="c", subcore_axis_name="s")
_SC_PARAMS = pltpu.CompilerParams(use_tc_tiling_on_sc=False,
                                  needs_layout_passes=False)


def _fill2d(ref, nrows, ncols, val):
    nv = ncols // 16

    def body(i, carry):
        r = i // nv
        k = i % nv
        ref[r, pl.ds(k * 16, 16)] = jnp.full((16,), val, f32)
        return carry

    lax.fori_loop(0, nrows * nv, body, 0)


@functools.partial(
    pl.kernel,
    out_type=[jax.ShapeDtypeStruct((32, NS, VCH, 128), i32),
              jax.ShapeDtypeStruct((32, 16), i32)],
    mesh=_MESH,
    compiler_params=_SC_PARAMS,
    scratch_types=[
        pltpu.VMEM((KR,), i32),
        pltpu.VMEM((VCH, 128), i32),
        pltpu.VMEM((VCH, 128), i32),
        pltpu.VMEM((16,), i32),
        pltpu.SemaphoreType.DMA,
    ],
)
def _k0(key3, part_out, tot_out, table_v, keyb_v, outb_v, tot_v, sem):
    c = lax.axis_index("c")
    s = lax.axis_index("s")
    w = c * NS + s
    lo = w * KR

    def zb(idx, carry):
        table_v[pl.ds(idx * 16, 16)] = jnp.zeros((16,), i32)
        return carry

    lax.fori_loop(0, KR // 16, zb, 0)

    ones16 = jnp.ones((16,), i32)

    def tpass(t, carry):
        pltpu.sync_copy(key3.at[t], keyb_v)

        def jloop(idx, carry2):
            j = idx // 8
            k = idx % 8
            vec = keyb_v[j, pl.ds(k * 16, 16)]
            rel = vec - lo
            m = (rel >= 0) & (rel < KR)
            plsc.store_scatter(table_v, [rel], ones16, mask=m)
            return carry2

        lax.fori_loop(0, VCH * 8, jloop, 0)
        return carry

    lax.fori_loop(0, NS, tpass, 0)

    def scan(idx, carry):
        v = table_v[pl.ds(idx * 16, 16)]
        inc = plsc.cumsum(v)
        table_v[pl.ds(idx * 16, 16)] = inc - v + carry
        return carry + jnp.sum(v)

    tot = lax.fori_loop(0, KR // 16, scan, i32(0))

    def qpass(t, carry):
        pltpu.sync_copy(key3.at[t], keyb_v)

        def jloop(idx, carry2):
            j = idx // 8
            k = idx % 8
            vec = keyb_v[j, pl.ds(k * 16, 16)]
            rel = vec - lo
            m = (rel >= 0) & (rel < KR)
            g = plsc.load_gather(table_v, [rel], mask=m)
            outb_v[j, pl.ds(k * 16, 16)] = jnp.where(m, g + 1, 0)
            return carry2

        lax.fori_loop(0, VCH * 8, jloop, 0)
        pltpu.sync_copy(outb_v, part_out.at[w, t])
        return carry

    lax.fori_loop(0, NS, qpass, 0)
    tot_v[pl.ds(0, 16)] = jnp.zeros((16,), i32) + tot
    pltpu.sync_copy(tot_v, tot_out.at[w])


@functools.partial(
    pl.kernel,
    out_type=jax.ShapeDtypeStruct((N_VOX, C), f32),
    mesh=_MESH,
    compiler_params=_SC_PARAMS,
    scratch_types=[
        pltpu.VMEM((KCH, 128), i32),
        pltpu.VMEM((128, 128), f32),
        pltpu.VMEM((128, 128), f32),
        pltpu.SemaphoreType.DMA,
        pltpu.SemaphoreType.DMA,
    ],
)
def _k2(B, invK, D_out, idx_v, rows_v, rows2_v, sem, sem2):
    c = lax.axis_index("c")
    s = lax.axis_index("s")
    w = c * NS + s
    base = w * KPT
    pltpu.sync_copy(invK.at[c, s], idx_v)

    nfull = KCH - 1
    pltpu.async_copy(B.at[idx_v.at[0]], rows_v, sem)

    def spair(jj, carry):
        j = 2 * jj
        pltpu.async_copy(B.at[idx_v.at[j + 1]], rows2_v, sem2)
        pltpu.make_async_copy(B.at[idx_v.at[j]], rows_v, sem).wait()
        pltpu.sync_copy(rows_v, D_out.at[pl.ds(base + j * 128, 128)])

        @pl.when(j + 2 < nfull)
        def _():
            pltpu.async_copy(B.at[idx_v.at[j + 2]], rows_v, sem)

        pltpu.make_async_copy(B.at[idx_v.at[j + 1]], rows2_v, sem2).wait()
        pltpu.sync_copy(rows2_v, D_out.at[pl.ds(base + (j + 1) * 128, 128)])
        return carry

    lax.fori_loop(0, nfull // 2, spair, 0)
    pltpu.async_copy(B.at[idx_v.at[KCH - 1]], rows_v, sem).wait()
    pltpu.sync_copy(rows_v.at[pl.ds(0, 53)],
                    D_out.at[pl.ds(base + (KCH - 1) * 128, 53)])


@functools.partial(
    pl.kernel,
    out_type=jax.ShapeDtypeStruct((4, PACC, 32), f32),
    mesh=_MESH,
    compiler_params=_SC_PARAMS,
    scratch_types=[
        pltpu.VMEM((PCH, 128), i32),
        pltpu.VMEM((PCH, 128), i32),
        pltpu.VMEM((128, 32), f32),
        pltpu.VMEM((128, 32), f32),
        pltpu.VMEM((56, 32), f32),
        pltpu.VMEM_SHARED((PACC, 32), f32),
        pltpu.SemaphoreType.DMA,
        pltpu.SemaphoreType.DMA,
    ],
)
def _k3(y4, gidx4, sidx, num_out,
        gidx_v, sidx_v, rows_v, rows2_v, z32_v, acc_sh, sem, sem2):
    c = lax.axis_index("c")
    s = lax.axis_index("s")
    spt = PACC // NS

    _fill2d(z32_v, 56, 32, 0.0)
    pltpu.sync_copy(sidx.at[s], sidx_v)

    for q in range(2):
        pltpu.sync_copy(gidx4.at[c, q, s], gidx_v)

        def zchunk(j, carry):
            pltpu.sync_copy(z32_v, acc_sh.at[pl.ds(s * spt + j * 56, 56)])
            return carry

        lax.fori_loop(0, spt // 56, zchunk, 0)
        plsc.subcore_barrier()

        pltpu.async_copy(y4.at[gidx_v.at[0]], rows_v, sem)

        def spair(jj, carry):
            j = 2 * jj
            pltpu.async_copy(y4.at[gidx_v.at[j + 1]], rows2_v, sem2)
            pltpu.make_async_copy(y4.at[gidx_v.at[j]], rows_v, sem).wait()
            pltpu.sync_copy(rows_v, acc_sh.at[sidx_v.at[j]], add=True)

            @pl.when(j + 2 < PCH)
            def _():
                pltpu.async_copy(y4.at[gidx_v.at[j + 2]], rows_v, sem)

            pltpu.make_async_copy(y4.at[gidx_v.at[j + 1]], rows2_v,
                                  sem2).wait()
            pltpu.sync_copy(rows2_v, acc_sh.at[sidx_v.at[j + 1]], add=True)
            return carry

        lax.fori_loop(0, PCH // 2, spair, 0)
        plsc.subcore_barrier()

        pltpu.sync_copy(acc_sh.at[pl.ds(s * spt, spt)],
                        num_out.at[2 * c + q, pl.ds(s * spt, spt)])


def kernel(features, coors, coors_inv, scale_coors_inv, W_in, b_in, W_pp1,
           b_pp1, g1, be1, W_pp2, b_pp2, g2, be2, W_pp3, b_pp3, W_out1,
           b_out1, W_out2, b_out2):
    key = (coors[:, 0] << 15) | ((coors[:, 1] >> 1) << 10) \
        | ((coors[:, 2] >> 1) << 5) | (coors[:, 3] >> 1)
    key3 = jnp.pad(key.reshape(NS, VPT), ((0, 0), (0, VCH * 128 - VPT)),
                   mode="edge").reshape(NS, VCH, 128)
    part, totals = _k0(key3)
    tot = totals[:, 0]
    offs = jnp.concatenate([jnp.zeros((1,), i32), jnp.cumsum(tot)])[:32]
    psum = jnp.sum(part, axis=0).reshape(NS, VCH * 128)[:, :VPT].reshape(-1)
    inv = psum - 1 + offs[key >> 16]
    n_valid = jnp.sum(tot).astype(f32)

    invK = jnp.pad(inv.reshape(32, KPT), ((0, 0), (0, KCH * 128 - KPT)),
                   constant_values=0).reshape(2, NS, KCH, 128)
    base4 = 4 * jnp.pad(coors_inv.reshape(NS, PPT),
                        ((0, 0), (0, PACC - PPT)),
                        constant_values=0).reshape(NS, PCH, 128)
    gidx4 = jnp.stack([jnp.stack([base4, base4 + 1]),
                       jnp.stack([base4 + 2, base4 + 3])])
    sidx = jnp.pad(scale_coors_inv.reshape(NS, PPT),
                   ((0, 0), (0, PACC - PPT)),
                   constant_values=DUMP3).reshape(NS, PCH, 128)
    cnt2 = jnp.zeros((N_COARSE,), f32).at[scale_coors_inv].add(
        1.0, mode="drop")

    W1a = W_out1[:C]
    W1b = W_out1[C:]

    F1, A = pl.pallas_call(
        _p0_body,
        grid=(GRID,),
        in_specs=[_rows(), _full((C, H)), _full((C, C)), _full((1, C)),
                  _full((C, C))],
        out_specs=[_rows(H), _rows()],
        out_shape=[jax.ShapeDtypeStruct((N_VOX, H), f32),
                   jax.ShapeDtypeStruct((N_VOX, C), f32)],
        compiler_params=_SEQ,
    )(features, W_pp1, W_in, b_in.reshape(1, C), W1a)

    fea_ext = jnp.concatenate([F1, jnp.ones((N_VOX, 8), f32)], axis=1)
    dsx = jnp.zeros((N_VOX, H + 8), f32).at[inv].add(fea_ext, mode="drop")

    h1, st1 = pl.pallas_call(
        _p1_body,
        grid=(GRID,),
        in_specs=[_rows(H + 8), _full((1, H))],
        out_specs=[_rows(H), _full((2, H))],
        out_shape=[jax.ShapeDtypeStruct((N_VOX, H), f32),
                   jax.ShapeDtypeStruct((2, H), f32)],
        compiler_params=_SEQ,
    )(dsx, b_pp1.reshape(1, H))

    n_empty = jnp.float32(N_VOX) - n_valid
    e1 = _leaky(b_pp1)
    m1 = (st1[0] - n_empty * e1) / n_valid
    q1 = (st1[1] - n_empty * e1 * e1) / n_valid
    a1 = g1 / jnp.sqrt(jnp.maximum(q1 - m1 * m1, 0.0) + 1e-5)
    c1 = be1 - m1 * a1
    W2f = a1[:, None] * W_pp2
    b2f = (c1 @ W_pp2 + b_pp2).reshape(1, H)

    h2, st2 = pl.pallas_call(
        _p2_body,
        grid=(GRID,),
        in_specs=[_rows(H), _full((H, H)), _full((1, H))],
        out_specs=[_rows(H), _full((2, H))],
        out_shape=[jax.ShapeDtypeStruct((N_VOX, H), f32),
                   jax.ShapeDtypeStruct((2, H), f32)],
        compiler_params=_SEQ,
    )(h1, W2f, b2f)

    e2 = _leaky(b2f[0])
    m2 = (st2[0] - n_empty * e2) / n_valid
    q2 = (st2[1] - n_empty * e2 * e2) / n_valid
    a2 = g2 / jnp.sqrt(jnp.maximum(q2 - m2 * m2, 0.0) + 1e-5)
    c2 = be2 - m2 * a2
    W3f = a2[:, None] * W_pp3
    b3f = (c2 @ W_pp3 + b_pp3).reshape(1, C)

    B = pl.pallas_call(
        _p3_body,
        grid=(GRID,),
        in_specs=[_rows(H), _full((H, C)), _full((1, C)), _full((C, C))],
        out_specs=_rows(),
        out_shape=jax.ShapeDtypeStruct((N_VOX, C), f32),
        compiler_params=_SEQ,
    )(h2, W3f, b3f, W1b)

    D = _k2(B, invK)

    y = pl.pallas_call(
        _p4_body,
        grid=(GRID,),
        in_specs=[_rows(), _rows(), _full((1, C)), _full((C, C)),
                  _full((1, C))],
        out_specs=_rows(),
        out_shape=jax.ShapeDtypeStruct((N_VOX, C), f32),
        compiler_params=_SEQ,
    )(A, D, b_out1.reshape(1, C), W_out2, b_out2.reshape(1, C))

    num = _k3(y.reshape(4 * N_VOX, 32), gidx4, sidx)
    scale = 1.0 / jnp.maximum(cnt2, 1.0)
    v_feat = jnp.concatenate(
        [num[0, :N_COARSE], num[1, :N_COARSE],
         num[2, :N_COARSE], num[3, :N_COARSE]], axis=1) * scale[:, None]
    return v_feat


# speedup vs baseline: 3.2894x; 1.1817x over previous
import functools

import jax
import jax.numpy as jnp
from jax import lax
from jax.experimental import pallas as pl
from jax.experimental.pallas import tpu as pltpu
from jax.experimental.pallas import tpu_sc as plsc

N_VOX = 100000
N_PTS = 400000
N_COARSE = 25000
C = 128
H = C // 2
KEYSPACE = 1 << 21
KR = KEYSPACE // 32

BR = 2000
GRID = N_VOX // BR

NS = 16

VCH = 49
VPT = 6250
KCH = 25
KPT = 3125
PCH = 196
PPT = 25000
PACC = PCH * 128
DUMP3 = N_COARSE

f32 = jnp.float32
i32 = jnp.int32


def _leaky(x):
    return jnp.where(x >= 0, x, 0.1 * x)


def _p0_body(feat_ref, Wpp1_ref, Win_ref, bin_ref, W1a_ref,
             F1_ref, A_ref):
    F1_ref[...] = jnp.dot(feat_ref[...], Wpp1_ref[...],
                          preferred_element_type=f32)
    idn = _leaky(jnp.dot(feat_ref[...], Win_ref[...],
                         preferred_element_type=f32) + bin_ref[...])
    A_ref[...] = jnp.dot(idn, W1a_ref[...], preferred_element_type=f32)


def _p1_body(dsx_ref, bpp1_ref, h1_ref, stats_ref):
    i = pl.program_id(0)
    x = dsx_ref[...]
    rc = jnp.maximum(x[:, H:H + 1], 1.0)
    h1 = _leaky(x[:, :H] / rc + bpp1_ref[...])
    h1_ref[...] = h1
    blk = jnp.concatenate([jnp.sum(h1, axis=0, keepdims=True),
                           jnp.sum(h1 * h1, axis=0, keepdims=True)], axis=0)

    @pl.when(i == 0)
    def _():
        stats_ref[...] = jnp.zeros_like(stats_ref)

    stats_ref[...] += blk


def _p2_body(h1_ref, W2_ref, b2_ref, h2_ref, stats_ref):
    i = pl.program_id(0)
    h2 = _leaky(jnp.dot(h1_ref[...], W2_ref[...],
                        preferred_element_type=f32) + b2_ref[...])
    h2_ref[...] = h2
    blk = jnp.concatenate([jnp.sum(h2, axis=0, keepdims=True),
                           jnp.sum(h2 * h2, axis=0, keepdims=True)], axis=0)

    @pl.when(i == 0)
    def _():
        stats_ref[...] = jnp.zeros_like(stats_ref)

    stats_ref[...] += blk


def _p3_body(h2_ref, W3_ref, b3_ref, W1b_ref, B_ref):
    h3 = _leaky(jnp.dot(h2_ref[...], W3_ref[...],
                        preferred_element_type=f32) + b3_ref[...])
    B_ref[...] = jnp.dot(h3, W1b_ref[...], preferred_element_type=f32)


def _p4_body(A_ref, D_ref, bo1_ref, Wo2_ref, bo2_ref, y_ref):
    pre = _leaky(A_ref[...] + D_ref[...] + bo1_ref[...])
    y_ref[...] = jnp.dot(pre, Wo2_ref[...],
                         preferred_element_type=f32) + bo2_ref[...]


def _rows(j=C):
    return pl.BlockSpec((BR, j), lambda i: (i, 0))


def _full(shape):
    return pl.BlockSpec(shape, lambda i: tuple(0 for _ in shape))


_SEQ = pltpu.CompilerParams(dimension_semantics=("arbitrary",))


_MESH = plsc.VectorSubcoreMesh(core_axis_name="c", subcore_axis_name="s")
_SC_PARAMS = pltpu.CompilerParams(use_tc_tiling_on_sc=False,
                                  needs_layout_passes=False)


def _fill2d(ref, nrows, ncols, val):
    nv = ncols // 16

    def body(i, carry):
        r = i // nv
        k = i % nv
        ref[r, pl.ds(k * 16, 16)] = jnp.full((16,), val, f32)
        return carry

    lax.fori_loop(0, nrows * nv, body, 0)


@functools.partial(
    pl.kernel,
    out_type=[jax.ShapeDtypeStruct((32, NS, VCH, 128), i32),
              jax.ShapeDtypeStruct((32, 16), i32)],
    mesh=_MESH,
    compiler_params=_SC_PARAMS,
    scratch_types=[
        pltpu.VMEM((KR,), i32),
        pltpu.VMEM((VCH, 128), i32),
        pltpu.VMEM((VCH, 128), i32),
        pltpu.VMEM((16,), i32),
        pltpu.SemaphoreType.DMA,
    ],
)
def _k0(key3, part_out, tot_out, table_v, keyb_v, outb_v, tot_v, sem):
    c = lax.axis_index("c")
    s = lax.axis_index("s")
    w = c * NS + s
    lo = w * KR

    def zb(idx, carry):
        table_v[pl.ds(idx * 16, 16)] = jnp.zeros((16,), i32)
        return carry

    lax.fori_loop(0, KR // 16, zb, 0)

    ones16 = jnp.ones((16,), i32)

    def tpass(t, carry):
        pltpu.sync_copy(key3.at[t], keyb_v)

        def jloop(idx, carry2):
            j = idx // 8
            k = idx % 8
            vec = keyb_v[j, pl.ds(k * 16, 16)]
            rel = vec - lo
            m = (rel >= 0) & (rel < KR)
            plsc.store_scatter(table_v, [rel], ones16, mask=m)
            return carry2

        lax.fori_loop(0, VCH * 8, jloop, 0)
        return carry

    lax.fori_loop(0, NS, tpass, 0)

    def scan(idx, carry):
        v = table_v[pl.ds(idx * 16, 16)]
        inc = plsc.cumsum(v)
        table_v[pl.ds(idx * 16, 16)] = inc - v + carry
        return carry + jnp.sum(v)

    tot = lax.fori_loop(0, KR // 16, scan, i32(0))

    def qpass(t, carry):
        pltpu.sync_copy(key3.at[t], keyb_v)

        def jloop(idx, carry2):
            j = idx // 8
            k = idx % 8
            vec = keyb_v[j, pl.ds(k * 16, 16)]
            rel = vec - lo
            m = (rel >= 0) & (rel < KR)
            g = plsc.load_gather(table_v, [rel], mask=m)
            outb_v[j, pl.ds(k * 16, 16)] = jnp.where(m, g + 1, 0)
            return carry2

        lax.fori_loop(0, VCH * 8, jloop, 0)
        pltpu.sync_copy(outb_v, part_out.at[w, t])
        return carry

    lax.fori_loop(0, NS, qpass, 0)
    tot_v[pl.ds(0, 16)] = jnp.zeros((16,), i32) + tot
    pltpu.sync_copy(tot_v, tot_out.at[w])


@functools.partial(
    pl.kernel,
    out_type=jax.ShapeDtypeStruct((N_VOX, C), f32),
    mesh=_MESH,
    compiler_params=_SC_PARAMS,
    scratch_types=[
        pltpu.VMEM((KCH, 128), i32),
        pltpu.VMEM((128, 128), f32),
        pltpu.VMEM((128, 128), f32),
        pltpu.SemaphoreType.DMA,
        pltpu.SemaphoreType.DMA,
    ],
)
def _k2(B, invK, D_out, idx_v, rows_v, rows2_v, sem, sem2):
    c = lax.axis_index("c")
    s = lax.axis_index("s")
    w = c * NS + s
    base = w * KPT
    pltpu.sync_copy(invK.at[c, s], idx_v)

    nfull = KCH - 1
    pltpu.async_copy(B.at[idx_v.at[0]], rows_v, sem)

    def spair(jj, carry):
        j = 2 * jj
        pltpu.async_copy(B.at[idx_v.at[j + 1]], rows2_v, sem2)
        pltpu.make_async_copy(B.at[idx_v.at[j]], rows_v, sem).wait()
        pltpu.sync_copy(rows_v, D_out.at[pl.ds(base + j * 128, 128)])

        @pl.when(j + 2 < nfull)
        def _():
            pltpu.async_copy(B.at[idx_v.at[j + 2]], rows_v, sem)

        pltpu.make_async_copy(B.at[idx_v.at[j + 1]], rows2_v, sem2).wait()
        pltpu.sync_copy(rows2_v, D_out.at[pl.ds(base + (j + 1) * 128, 128)])
        return carry

    lax.fori_loop(0, nfull // 2, spair, 0)
    pltpu.async_copy(B.at[idx_v.at[KCH - 1]], rows_v, sem).wait()
    pltpu.sync_copy(rows_v.at[pl.ds(0, 53)],
                    D_out.at[pl.ds(base + (KCH - 1) * 128, 53)])


@functools.partial(
    pl.kernel,
    out_type=jax.ShapeDtypeStruct((6, PACC, 32), f32),
    mesh=_MESH,
    compiler_params=_SC_PARAMS,
    scratch_types=[
        pltpu.VMEM((PCH, 128), i32),
        pltpu.VMEM((PCH, 128), i32),
        pltpu.VMEM((128, 32), f32),
        pltpu.VMEM((128, 32), f32),
        pltpu.VMEM((128, 32), f32),
        pltpu.VMEM((56, 32), f32),
        pltpu.VMEM_SHARED((PACC, 32), f32),
        pltpu.SemaphoreType.DMA,
        pltpu.SemaphoreType.DMA,
    ],
)
def _k3(y4, gidx4, sidx, num_out,
        gidx_v, sidx_v, rows_v, rows2_v, ones_v, z32_v, acc_sh, sem, sem2):
    c = lax.axis_index("c")
    s = lax.axis_index("s")
    spt = PACC // NS

    _fill2d(z32_v, 56, 32, 0.0)
    _fill2d(ones_v, 128, 32, 1.0)
    pltpu.sync_copy(sidx.at[s], sidx_v)

    for q in range(2):
        pltpu.sync_copy(gidx4.at[c, q, s], gidx_v)

        def zchunk(j, carry):
            pltpu.sync_copy(z32_v, acc_sh.at[pl.ds(s * spt + j * 56, 56)])
            return carry

        lax.fori_loop(0, spt // 56, zchunk, 0)
        plsc.subcore_barrier()

        pltpu.async_copy(y4.at[gidx_v.at[0]], rows_v, sem)

        def spair(jj, carry):
            j = 2 * jj
            pltpu.async_copy(y4.at[gidx_v.at[j + 1]], rows2_v, sem2)
            pltpu.make_async_copy(y4.at[gidx_v.at[j]], rows_v, sem).wait()
            pltpu.sync_copy(rows_v, acc_sh.at[sidx_v.at[j]], add=True)

            @pl.when(j + 2 < PCH)
            def _():
                pltpu.async_copy(y4.at[gidx_v.at[j + 2]], rows_v, sem)

            pltpu.make_async_copy(y4.at[gidx_v.at[j + 1]], rows2_v,
                                  sem2).wait()
            pltpu.sync_copy(rows2_v, acc_sh.at[sidx_v.at[j + 1]], add=True)
            return carry

        lax.fori_loop(0, PCH // 2, spair, 0)
        plsc.subcore_barrier()

        pltpu.sync_copy(acc_sh.at[pl.ds(s * spt, spt)],
                        num_out.at[2 * c + q, pl.ds(s * spt, spt)])

    def czchunk(j, carry):
        pltpu.sync_copy(z32_v, acc_sh.at[pl.ds(s * spt + j * 56, 56)])
        return carry

    lax.fori_loop(0, spt // 56, czchunk, 0)
    plsc.subcore_barrier()

    half = PCH // 2

    def cchunk(j, carry):
        pltpu.sync_copy(ones_v, acc_sh.at[sidx_v.at[c * half + j]], add=True)
        return carry

    lax.fori_loop(0, half, cchunk, 0)
    plsc.subcore_barrier()
    pltpu.sync_copy(acc_sh.at[pl.ds(s * spt, spt)],
                    num_out.at[4 + c, pl.ds(s * spt, spt)])


def kernel(features, coors, coors_inv, scale_coors_inv, W_in, b_in, W_pp1,
           b_pp1, g1, be1, W_pp2, b_pp2, g2, be2, W_pp3, b_pp3, W_out1,
           b_out1, W_out2, b_out2):
    key = (coors[:, 0] << 15) | ((coors[:, 1] >> 1) << 10) \
        | ((coors[:, 2] >> 1) << 5) | (coors[:, 3] >> 1)
    key3 = jnp.pad(key.reshape(NS, VPT), ((0, 0), (0, VCH * 128 - VPT)),
                   mode="edge").reshape(NS, VCH, 128)
    part, totals = _k0(key3)
    tot = totals[:, 0]
    offs = jnp.concatenate([jnp.zeros((1,), i32), jnp.cumsum(tot)])[:32]
    psum = jnp.sum(part, axis=0).reshape(NS, VCH * 128)[:, :VPT].reshape(-1)
    inv = psum - 1 + offs[key >> 16]
    n_valid = jnp.sum(tot).astype(f32)

    invK = jnp.pad(inv.reshape(32, KPT), ((0, 0), (0, KCH * 128 - KPT)),
                   constant_values=0).reshape(2, NS, KCH, 128)
    base4 = 4 * jnp.pad(coors_inv.reshape(NS, PPT),
                        ((0, 0), (0, PACC - PPT)),
                        constant_values=0).reshape(NS, PCH, 128)
    gidx4 = jnp.stack([jnp.stack([base4, base4 + 1]),
                       jnp.stack([base4 + 2, base4 + 3])])
    sidx = jnp.pad(scale_coors_inv.reshape(NS, PPT),
                   ((0, 0), (0, PACC - PPT)),
                   constant_values=DUMP3).reshape(NS, PCH, 128)

    W1a = W_out1[:C]
    W1b = W_out1[C:]

    F1, A = pl.pallas_call(
        _p0_body,
        grid=(GRID,),
        in_specs=[_rows(), _full((C, H)), _full((C, C)), _full((1, C)),
                  _full((C, C))],
        out_specs=[_rows(H), _rows()],
        out_shape=[jax.ShapeDtypeStruct((N_VOX, H), f32),
                   jax.ShapeDtypeStruct((N_VOX, C), f32)],
        compiler_params=_SEQ,
    )(features, W_pp1, W_in, b_in.reshape(1, C), W1a)

    fea_ext = jnp.concatenate([F1, jnp.ones((N_VOX, 8), f32)], axis=1)
    dsx = jnp.zeros((N_VOX, H + 8), f32).at[inv].add(fea_ext, mode="drop")

    h1, st1 = pl.pallas_call(
        _p1_body,
        grid=(GRID,),
        in_specs=[_rows(H + 8), _full((1, H))],
        out_specs=[_rows(H), _full((2, H))],
        out_shape=[jax.ShapeDtypeStruct((N_VOX, H), f32),
                   jax.ShapeDtypeStruct((2, H), f32)],
        compiler_params=_SEQ,
    )(dsx, b_pp1.reshape(1, H))

    n_empty = jnp.float32(N_VOX) - n_valid
    e1 = _leaky(b_pp1)
    m1 = (st1[0] - n_empty * e1) / n_valid
    q1 = (st1[1] - n_empty * e1 * e1) / n_valid
    a1 = g1 / jnp.sqrt(jnp.maximum(q1 - m1 * m1, 0.0) + 1e-5)
    c1 = be1 - m1 * a1
    W2f = a1[:, None] * W_pp2
    b2f = (c1 @ W_pp2 + b_pp2).reshape(1, H)

    h2, st2 = pl.pallas_call(
        _p2_body,
        grid=(GRID,),
        in_specs=[_rows(H), _full((H, H)), _full((1, H))],
        out_specs=[_rows(H), _full((2, H))],
        out_shape=[jax.ShapeDtypeStruct((N_VOX, H), f32),
                   jax.ShapeDtypeStruct((2, H), f32)],
        compiler_params=_SEQ,
    )(h1, W2f, b2f)

    e2 = _leaky(b2f[0])
    m2 = (st2[0] - n_empty * e2) / n_valid
    q2 = (st2[1] - n_empty * e2 * e2) / n_valid
    a2 = g2 / jnp.sqrt(jnp.maximum(q2 - m2 * m2, 0.0) + 1e-5)
    c2 = be2 - m2 * a2
    W3f = a2[:, None] * W_pp3
    b3f = (c2 @ W_pp3 + b_pp3).reshape(1, C)

    B = pl.pallas_call(
        _p3_body,
        grid=(GRID,),
        in_specs=[_rows(H), _full((H, C)), _full((1, C)), _full((C, C))],
        out_specs=_rows(),
        out_shape=jax.ShapeDtypeStruct((N_VOX, C), f32),
        compiler_params=_SEQ,
    )(h2, W3f, b3f, W1b)

    D = _k2(B, invK)

    y = pl.pallas_call(
        _p4_body,
        grid=(GRID,),
        in_specs=[_rows(), _rows(), _full((1, C)), _full((C, C)),
                  _full((1, C))],
        out_specs=_rows(),
        out_shape=jax.ShapeDtypeStruct((N_VOX, C), f32),
        compiler_params=_SEQ,
    )(A, D, b_out1.reshape(1, C), W_out2, b_out2.reshape(1, C))

    num = _k3(y.reshape(4 * N_VOX, 32), gidx4, sidx)
    cnt2 = num[4, :N_COARSE, 0] + num[5, :N_COARSE, 0]
    scale = 1.0 / jnp.maximum(cnt2, 1.0)
    v_feat = jnp.concatenate(
        [num[0, :N_COARSE], num[1, :N_COARSE],
         num[2, :N_COARSE], num[3, :N_COARSE]], axis=1) * scale[:, None]
    return v_feat
